# baseline (speedup 1.0000x reference)
import functools

import jax
import jax.numpy as jnp
from jax import lax
from jax.experimental import pallas as pl
from jax.experimental.pallas import tpu as pltpu
from jax.experimental.pallas import tpu_sc as plsc

U = 1024
DIM = 64
DEPTH = 16

B = 4096 * 200
IDXW = 128
NC, NS = 2, 16
NW = NC * NS
ROWS_PER_W = B // IDXW // NW
CHUNK = 4
NCHUNK = ROWS_PER_W // CHUNK
NPAIR = NCHUNK // 2


def _embed_body(unique_ref, prim_ref, out_ref):
    u = unique_ref[:]
    m0 = prim_ref[0]
    m1 = prim_ref[1]
    maps = jnp.ones((U, DIM), jnp.float32)
    dn = (((1,), (1,)), ((), ()))
    for depth in range(DEPTH):
        shifted = u >> depth
        code = jnp.where(shifted > 1, shifted & 1, 2)
        a = lax.dot_general(maps, m0, dn, preferred_element_type=jnp.float32)
        b = lax.dot_general(maps, m1, dn, preferred_element_type=jnp.float32)
        maps = jnp.where(code == 0, a, jnp.where(code == 1, b, maps))
    out_ref[:] = maps


def _embed(unique, primitives):
    return pl.pallas_call(
        _embed_body,
        out_shape=jax.ShapeDtypeStruct((U, DIM), jnp.float32),
    )(unique.reshape(U, 1), primitives)


def _gather_body(table_hbm, idx_hbm, out_hbm,
                 idx_all, rows0, rows1, gs0, gs1, os0, os1):
    wid = lax.axis_index("s") * NC + lax.axis_index("c")
    base = wid * ROWS_PER_W

    pltpu.sync_copy(idx_hbm.at[pl.ds(base, ROWS_PER_W)], idx_all)

    def fire_gathers(rows, gsem, c):
        for j in range(CHUNK):
            pltpu.async_copy(
                table_hbm.at[idx_all.at[c * CHUNK + j]],
                rows.at[pl.ds(j * IDXW, IDXW)],
                gsem,
            )

    def wait_gathers(rows, gsem):
        for j in range(CHUNK):
            pltpu.make_async_copy(
                table_hbm.at[idx_all.at[j]],
                rows.at[pl.ds(j * IDXW, IDXW)],
                gsem,
            ).wait()

    def fire_out(rows, osem, c):
        pltpu.async_copy(
            rows, out_hbm.at[pl.ds((base + c * CHUNK) * IDXW, CHUNK * IDXW)],
            osem,
        )

    def wait_out(rows, osem):
        pltpu.make_async_copy(
            rows, out_hbm.at[pl.ds(base * IDXW, CHUNK * IDXW)], osem,
        ).wait()

    fire_gathers(rows0, gs0, 0)
    fire_gathers(rows1, gs1, 1)

    def pair(p, _):
        c0 = 2 * p
        wait_gathers(rows0, gs0)
        fire_out(rows0, os0, c0)
        wait_gathers(rows1, gs1)
        fire_out(rows1, os1, c0 + 1)

        @pl.when(p < NPAIR - 1)
        def _():
            wait_out(rows0, os0)
            fire_gathers(rows0, gs0, c0 + 2)
            wait_out(rows1, os1)
            fire_gathers(rows1, gs1, c0 + 3)
        return ()

    lax.fori_loop(0, NPAIR, pair, (), unroll=False)
    wait_out(rows0, os0)
    wait_out(rows1, os1)


@functools.partial(jax.jit, static_argnums=())
def _gather(table, idx2d):
    mesh = plsc.VectorSubcoreMesh(core_axis_name="c", subcore_axis_name="s")
    f = pl.kernel(
        _gather_body,
        out_type=jax.ShapeDtypeStruct((B, DIM), jnp.float32),
        mesh=mesh,
        scratch_types=[
            pltpu.VMEM((ROWS_PER_W, IDXW), jnp.int32),
            pltpu.VMEM((CHUNK * IDXW, DIM), jnp.float32),
            pltpu.VMEM((CHUNK * IDXW, DIM), jnp.float32),
            pltpu.SemaphoreType.DMA,
            pltpu.SemaphoreType.DMA,
            pltpu.SemaphoreType.DMA,
            pltpu.SemaphoreType.DMA,
        ],
        compiler_params=pltpu.CompilerParams(use_tc_tiling_on_sc=False),
    )
    return f(table, idx2d)


def kernel(unique, mapping, primitives):
    table = _embed(unique, primitives)
    idx2d = mapping.reshape(B // IDXW, IDXW)
    out = _gather(table, idx2d)
    return out.reshape(*mapping.shape, DIM)


# speedup vs baseline: 1.0012x; 1.0012x over previous
import functools

import jax
import jax.numpy as jnp
from jax import lax
from jax.experimental import pallas as pl
from jax.experimental.pallas import tpu as pltpu
from jax.experimental.pallas import tpu_sc as plsc

U = 1024
DIM = 64
DEPTH = 16

BATCH = 4096
SEQ = 200
S0, S1 = 128, 72
NC, NS = 2, 16
NW = NC * NS
B_PER_W = BATCH // NW
CB = 2
NCHUNK = B_PER_W // CB
NPAIR = NCHUNK // 2


def _embed_body(unique_ref, prim_ref, out_ref):
    u = unique_ref[:]
    m0 = prim_ref[0]
    m1 = prim_ref[1]
    maps = jnp.ones((U, DIM), jnp.float32)
    dn = (((1,), (1,)), ((), ()))
    for depth in range(DEPTH):
        shifted = u >> depth
        code = jnp.where(shifted > 1, shifted & 1, 2)
        a = lax.dot_general(maps, m0, dn, preferred_element_type=jnp.float32)
        b = lax.dot_general(maps, m1, dn, preferred_element_type=jnp.float32)
        maps = jnp.where(code == 0, a, jnp.where(code == 1, b, maps))
    out_ref[:] = maps


def _embed(unique, primitives):
    return pl.pallas_call(
        _embed_body,
        out_shape=jax.ShapeDtypeStruct((U, DIM), jnp.float32),
    )(unique.reshape(U, 1), primitives)


def _gather_body(map_hbm, table_hbm, out_hbm,
                 idx_all, rows0, rows1, gs0, gs1, os0, os1):
    wid = lax.axis_index("s") * NC + lax.axis_index("c")
    base_b = wid * B_PER_W

    pltpu.sync_copy(map_hbm.at[pl.ds(base_b, B_PER_W)], idx_all)

    def fire_gathers(rows, gsem, c):
        for r in range(CB):
            b = c * CB + r
            pltpu.async_copy(
                table_hbm.at[idx_all.at[b, pl.ds(0, S0)]],
                rows.at[r, pl.ds(0, S0)], gsem)
            pltpu.async_copy(
                table_hbm.at[idx_all.at[b, pl.ds(S0, S1)]],
                rows.at[r, pl.ds(S0, S1)], gsem)

    def wait_gathers(rows, gsem):
        for r in range(CB):
            pltpu.make_async_copy(
                table_hbm.at[idx_all.at[0, pl.ds(0, S0)]],
                rows.at[r, pl.ds(0, S0)], gsem).wait()
            pltpu.make_async_copy(
                table_hbm.at[idx_all.at[0, pl.ds(S0, S1)]],
                rows.at[r, pl.ds(S0, S1)], gsem).wait()

    def fire_out(rows, osem, c):
        pltpu.async_copy(
            rows, out_hbm.at[pl.ds(base_b + c * CB, CB)], osem)

    def wait_out(rows, osem):
        pltpu.make_async_copy(
            rows, out_hbm.at[pl.ds(base_b, CB)], osem).wait()

    fire_gathers(rows0, gs0, 0)
    fire_gathers(rows1, gs1, 1)

    def pair(p, _):
        c0 = 2 * p
        wait_gathers(rows0, gs0)
        fire_out(rows0, os0, c0)
        wait_gathers(rows1, gs1)
        fire_out(rows1, os1, c0 + 1)

        @pl.when(p < NPAIR - 1)
        def _():
            wait_out(rows0, os0)
            fire_gathers(rows0, gs0, c0 + 2)
            wait_out(rows1, os1)
            fire_gathers(rows1, gs1, c0 + 3)
        return ()

    lax.fori_loop(0, NPAIR, pair, (), unroll=False)
    wait_out(rows0, os0)
    wait_out(rows1, os1)


@functools.partial(jax.jit, static_argnums=())
def _gather(mapping, table):
    mesh = plsc.VectorSubcoreMesh(core_axis_name="c", subcore_axis_name="s")
    f = pl.kernel(
        _gather_body,
        out_type=jax.ShapeDtypeStruct((BATCH, SEQ, DIM), jnp.float32),
        mesh=mesh,
        scratch_types=[
            pltpu.VMEM((B_PER_W, SEQ), jnp.int32),
            pltpu.VMEM((CB, SEQ, DIM), jnp.float32),
            pltpu.VMEM((CB, SEQ, DIM), jnp.float32),
            pltpu.SemaphoreType.DMA,
            pltpu.SemaphoreType.DMA,
            pltpu.SemaphoreType.DMA,
            pltpu.SemaphoreType.DMA,
        ],
        compiler_params=pltpu.CompilerParams(use_tc_tiling_on_sc=False),
    )
    return f(mapping, table)


def kernel(unique, mapping, primitives):
    table = _embed(unique, primitives)
    return _gather(mapping, table)


# speedup vs baseline: 1.5768x; 1.5750x over previous
import functools

import jax
import jax.numpy as jnp
from jax import lax
from jax.experimental import pallas as pl
from jax.experimental.pallas import tpu as pltpu
from jax.experimental.pallas import tpu_sc as plsc

U = 1024
DIM = 64
DEPTH = 16

BATCH = 4096
SEQ = 200
NC, NS = 2, 16
NW = NC * NS
LW = 128
NBG = LW // 16
NPAIR = SEQ // 2


def _embed_body(unique_ref, prim_ref, out_ref):
    u = unique_ref[:]
    m0 = prim_ref[0]
    m1 = prim_ref[1]
    mapsT = jnp.ones((DIM, U), jnp.float32)
    dn = (((1,), (0,)), ((), ()))
    for depth in range(DEPTH):
        shifted = u >> depth
        code = jnp.where(shifted > 1, shifted & 1, 2)
        a = lax.dot_general(m0, mapsT, dn, preferred_element_type=jnp.float32)
        b = lax.dot_general(m1, mapsT, dn, preferred_element_type=jnp.float32)
        mapsT = jnp.where(code == 0, a, jnp.where(code == 1, b, mapsT))
    out_ref[:] = mapsT


def _embed(unique, primitives):
    return pl.pallas_call(
        _embed_body,
        out_shape=jax.ShapeDtypeStruct((DIM, U), jnp.float32),
    )(unique.reshape(1, U), primitives)


def _gather_body(mapT_hbm, tableT_hbm, out_hbm,
                 table_v, idx_v, st0, st1, os0, os1):
    wid = lax.axis_index("s") * NC + lax.axis_index("c")
    lane0 = wid * LW

    pltpu.sync_copy(tableT_hbm, table_v)
    pltpu.sync_copy(mapT_hbm.at[:, pl.ds(lane0, LW)], idx_v)

    def compute(st, s):
        for bg in range(NBG):
            idx = idx_v[s, pl.ds(bg * 16, 16)]
            for d in range(DIM):
                v = plsc.load_gather(table_v.at[pl.ds(d * U, U)], [idx])
                st[0, d, pl.ds(bg * 16, 16)] = v

    def fire_out(st, osem, s):
        pltpu.async_copy(
            st, out_hbm.at[pl.ds(s, 1), :, pl.ds(lane0, LW)], osem)

    def wait_out(st, osem):
        pltpu.make_async_copy(
            st, out_hbm.at[pl.ds(0, 1), :, pl.ds(lane0, LW)], osem).wait()

    def pair(p, _):
        s0 = 2 * p

        @pl.when(p > 0)
        def _():
            wait_out(st0, os0)
        compute(st0, s0)
        fire_out(st0, os0, s0)

        @pl.when(p > 0)
        def _():
            wait_out(st1, os1)
        compute(st1, s0 + 1)
        fire_out(st1, os1, s0 + 1)
        return ()

    lax.fori_loop(0, NPAIR, pair, (), unroll=False)
    wait_out(st0, os0)
    wait_out(st1, os1)


@functools.partial(jax.jit, static_argnums=())
def _gather(mapT, tableT_flat):
    mesh = plsc.VectorSubcoreMesh(core_axis_name="c", subcore_axis_name="s")
    f = pl.kernel(
        _gather_body,
        out_type=jax.ShapeDtypeStruct((SEQ, DIM, BATCH), jnp.float32),
        mesh=mesh,
        scratch_types=[
            pltpu.VMEM((DIM * U,), jnp.float32),
            pltpu.VMEM((SEQ, LW), jnp.int32),
            pltpu.VMEM((1, DIM, LW), jnp.float32),
            pltpu.VMEM((1, DIM, LW), jnp.float32),
            pltpu.SemaphoreType.DMA,
            pltpu.SemaphoreType.DMA,
        ],
        compiler_params=pltpu.CompilerParams(
            use_tc_tiling_on_sc=True, needs_layout_passes=False),
    )
    return f(mapT, tableT_flat)


def kernel(unique, mapping, primitives):
    tableT = _embed(unique, primitives)
    outP = _gather(mapping.T, tableT.reshape(DIM * U))
    return jnp.transpose(outP, (2, 0, 1))


# speedup vs baseline: 4.1543x; 2.6346x over previous
import functools

import jax
import jax.numpy as jnp
from jax import lax
from jax.experimental import pallas as pl
from jax.experimental.pallas import tpu as pltpu
from jax.experimental.pallas import tpu_sc as plsc

U = 1024
DIM = 64
DEPTH = 16

BATCH = 4096
SEQ = 200
NC, NS = 2, 16
NW = NC * NS
LW = 128
NBG = LW // 16
NPAIR = SEQ // 2


def _embed_body(unique_ref, prim_ref, out_ref):
    u = unique_ref[:]
    m0 = prim_ref[0]
    m1 = prim_ref[1]
    mapsT = jnp.ones((DIM, U), jnp.float32)
    dn = (((1,), (0,)), ((), ()))
    for depth in range(DEPTH):
        shifted = u >> depth
        code = jnp.where(shifted > 1, shifted & 1, 2)
        a = lax.dot_general(m0, mapsT, dn, preferred_element_type=jnp.float32)
        b = lax.dot_general(m1, mapsT, dn, preferred_element_type=jnp.float32)
        mapsT = jnp.where(code == 0, a, jnp.where(code == 1, b, mapsT))
    out_ref[:] = mapsT


def _embed(unique, primitives):
    return pl.pallas_call(
        _embed_body,
        out_shape=jax.ShapeDtypeStruct((DIM, U), jnp.float32),
    )(unique.reshape(1, U), primitives)


def _gather_body(mapT_hbm, tableT_hbm, out_hbm,
                 table_v, idx_v, st0, st1, os0, os1):
    wid = lax.axis_index("s") * NC + lax.axis_index("c")
    lane0 = wid * LW

    pltpu.sync_copy(tableT_hbm, table_v)
    pltpu.sync_copy(mapT_hbm.at[:, pl.ds(lane0, LW)], idx_v)

    LAT = 6

    def compute(st, s):
        for bg in range(NBG):
            idx = idx_v[s, pl.ds(bg * 16, 16)]
            vals = {}
            for d in range(DIM + LAT):
                if d < DIM:
                    vals[d] = plsc.load_gather(
                        table_v.at[pl.ds(d * U, U)], [idx])
                if d >= LAT:
                    st[0, d - LAT, pl.ds(bg * 16, 16)] = vals.pop(d - LAT)

    def fire_out(st, osem, s):
        pltpu.async_copy(
            st, out_hbm.at[pl.ds(s, 1), :, pl.ds(lane0, LW)], osem)

    def wait_out(st, osem):
        pltpu.make_async_copy(
            st, out_hbm.at[pl.ds(0, 1), :, pl.ds(lane0, LW)], osem).wait()

    def pair(p, _):
        s0 = 2 * p

        @pl.when(p > 0)
        def _():
            wait_out(st0, os0)
        compute(st0, s0)
        fire_out(st0, os0, s0)

        @pl.when(p > 0)
        def _():
            wait_out(st1, os1)
        compute(st1, s0 + 1)
        fire_out(st1, os1, s0 + 1)
        return ()

    lax.fori_loop(0, NPAIR, pair, (), unroll=False)
    wait_out(st0, os0)
    wait_out(st1, os1)


@functools.partial(jax.jit, static_argnums=())
def _gather(mapT, tableT_flat):
    mesh = plsc.VectorSubcoreMesh(core_axis_name="c", subcore_axis_name="s")
    f = pl.kernel(
        _gather_body,
        out_type=jax.ShapeDtypeStruct((SEQ, DIM, BATCH), jnp.float32),
        mesh=mesh,
        scratch_types=[
            pltpu.VMEM((DIM * U,), jnp.float32),
            pltpu.VMEM((SEQ, LW), jnp.int32),
            pltpu.VMEM((1, DIM, LW), jnp.float32),
            pltpu.VMEM((1, DIM, LW), jnp.float32),
            pltpu.SemaphoreType.DMA,
            pltpu.SemaphoreType.DMA,
        ],
        compiler_params=pltpu.CompilerParams(
            use_tc_tiling_on_sc=True, needs_layout_passes=False),
    )
    return f(mapT, tableT_flat)


def kernel(unique, mapping, primitives):
    tableT = _embed(unique, primitives)
    outP = _gather(mapping.T, tableT.reshape(DIM * U))
    return jnp.transpose(outP, (2, 0, 1))
